# Initial kernel scaffold; baseline (speedup 1.0000x reference)
import jax
import jax.numpy as jnp
from jax.experimental import pallas as pl


def kernel(edge_types, table):
    raise NotImplementedError("write your pallas kernel here")


# speedup vs baseline: 6.1273x; 6.1273x over previous
import functools

import jax
import jax.numpy as jnp
from jax import lax
from jax.experimental import pallas as pl
from jax.experimental.pallas import tpu as pltpu
from jax.experimental.pallas import tpu_sc as plsc

ROWS, COLS = 16384, 200
EMBED_DIM = 32
B = ROWS * COLS

_info = plsc.get_sparse_core_info()
NC, NS = _info.num_cores, _info.num_subcores
NW = NC * NS
B_PER_W = B // NW
CHUNK = 1024
NCH = B_PER_W // CHUNK


@jax.jit
def _lookup(idx, table):
    mesh = plsc.VectorSubcoreMesh(core_axis_name="c", subcore_axis_name="s")

    @functools.partial(
        pl.kernel,
        mesh=mesh,
        out_type=jax.ShapeDtypeStruct((B, EMBED_DIM), jnp.float32),
        scratch_types=[
            pltpu.VMEM((CHUNK,), jnp.int32),
            pltpu.VMEM((CHUNK, EMBED_DIM), jnp.float32),
            pltpu.SemaphoreType.DMA,
        ],
        compiler_params=pltpu.CompilerParams(use_tc_tiling_on_sc=False),
    )
    def k(idx_hbm, table_hbm, out_hbm, idx_v, rows_v, sem):
        wid = lax.axis_index("s") * NC + lax.axis_index("c")
        base = wid * B_PER_W

        def body(g, carry):
            off = base + g * CHUNK
            pltpu.sync_copy(idx_hbm.at[pl.ds(off, CHUNK)], idx_v)
            pltpu.async_copy(table_hbm.at[idx_v], rows_v, sem).wait()
            pltpu.sync_copy(rows_v, out_hbm.at[pl.ds(off, CHUNK)])
            return carry

        lax.fori_loop(0, NCH, body, 0)

    return k(idx, table)


def kernel(edge_types, table):
    idx = edge_types.reshape(B).astype(jnp.int32)
    out = _lookup(idx, table)
    return out.reshape(ROWS, COLS, EMBED_DIM)


# speedup vs baseline: 6.4708x; 1.0561x over previous
import functools

import jax
import jax.numpy as jnp
from jax import lax
from jax.experimental import pallas as pl
from jax.experimental.pallas import tpu as pltpu
from jax.experimental.pallas import tpu_sc as plsc

ROWS, COLS = 16384, 200
EMBED_DIM = 32
B = ROWS * COLS

_info = plsc.get_sparse_core_info()
NC, NS = _info.num_cores, _info.num_subcores
NW = NC * NS
B_PER_W = B // NW
CHUNK = 1024
NCH = B_PER_W // CHUNK
NBUF = 2
NOUTER = NCH // NBUF


@jax.jit
def _lookup(idx, table):
    mesh = plsc.VectorSubcoreMesh(core_axis_name="c", subcore_axis_name="s")

    @functools.partial(
        pl.kernel,
        mesh=mesh,
        out_type=jax.ShapeDtypeStruct((B, EMBED_DIM), jnp.float32),
        scratch_types=(
            [pltpu.VMEM((CHUNK,), jnp.int32) for _ in range(NBUF)]
            + [pltpu.VMEM((CHUNK, EMBED_DIM), jnp.float32) for _ in range(NBUF)]
            + [pltpu.SemaphoreType.DMA for _ in range(3 * NBUF)]
        ),
        compiler_params=pltpu.CompilerParams(use_tc_tiling_on_sc=False),
    )
    def k(idx_hbm, table_hbm, out_hbm, *scratch):
        idx_bufs = scratch[0:NBUF]
        row_bufs = scratch[NBUF : 2 * NBUF]
        idx_sems = scratch[2 * NBUF : 3 * NBUF]
        gat_sems = scratch[3 * NBUF : 4 * NBUF]
        out_sems = scratch[4 * NBUF : 5 * NBUF]

        wid = lax.axis_index("s") * NC + lax.axis_index("c")
        base = wid * B_PER_W

        for b in range(NBUF):
            pltpu.async_copy(
                idx_hbm.at[pl.ds(base + b * CHUNK, CHUNK)], idx_bufs[b], idx_sems[b]
            )

        def body(go, carry):
            for b in range(NBUF):
                off = base + (go * NBUF + b) * CHUNK
                pltpu.make_async_copy(
                    idx_hbm.at[pl.ds(off, CHUNK)], idx_bufs[b], idx_sems[b]
                ).wait()

                @pl.when(go > 0)
                def _():
                    pltpu.make_async_copy(
                        row_bufs[b],
                        out_hbm.at[pl.ds(off - NBUF * CHUNK, CHUNK)],
                        out_sems[b],
                    ).wait()

                gat = pltpu.async_copy(
                    table_hbm.at[idx_bufs[b]], row_bufs[b], gat_sems[b]
                )
                gat.wait()

                @pl.when(go < NOUTER - 1)
                def _():
                    pltpu.async_copy(
                        idx_hbm.at[pl.ds(off + NBUF * CHUNK, CHUNK)],
                        idx_bufs[b],
                        idx_sems[b],
                    )

                pltpu.async_copy(row_bufs[b], out_hbm.at[pl.ds(off, CHUNK)], out_sems[b])
            return carry

        lax.fori_loop(0, NOUTER, body, 0)

        for b in range(NBUF):
            off = base + (NCH - NBUF + b) * CHUNK
            pltpu.make_async_copy(
                row_bufs[b], out_hbm.at[pl.ds(off, CHUNK)], out_sems[b]
            ).wait()

    return k(idx, table)


def kernel(edge_types, table):
    idx = edge_types.reshape(B).astype(jnp.int32)
    out = _lookup(idx, table)
    return out.reshape(ROWS, COLS, EMBED_DIM)


# speedup vs baseline: 6.4966x; 1.0040x over previous
import functools

import jax
import jax.numpy as jnp
from jax import lax
from jax.experimental import pallas as pl
from jax.experimental.pallas import tpu as pltpu
from jax.experimental.pallas import tpu_sc as plsc

ROWS, COLS = 16384, 200
EMBED_DIM = 32
B = ROWS * COLS

_info = plsc.get_sparse_core_info()
NC, NS = _info.num_cores, _info.num_subcores
NW = NC * NS
B_PER_W = B // NW
CHUNK = 640
NCH = B_PER_W // CHUNK
NBUF = 4
NOUTER = NCH // NBUF


@jax.jit
def _lookup(idx, table):
    mesh = plsc.VectorSubcoreMesh(core_axis_name="c", subcore_axis_name="s")

    @functools.partial(
        pl.kernel,
        mesh=mesh,
        out_type=jax.ShapeDtypeStruct((B, EMBED_DIM), jnp.float32),
        scratch_types=(
            [pltpu.VMEM((CHUNK,), jnp.int32) for _ in range(NBUF)]
            + [pltpu.VMEM((CHUNK, EMBED_DIM), jnp.float32) for _ in range(NBUF)]
            + [pltpu.SemaphoreType.DMA for _ in range(3 * NBUF)]
        ),
        compiler_params=pltpu.CompilerParams(use_tc_tiling_on_sc=False),
    )
    def k(idx_hbm, table_hbm, out_hbm, *scratch):
        idx_bufs = scratch[0:NBUF]
        row_bufs = scratch[NBUF : 2 * NBUF]
        idx_sems = scratch[2 * NBUF : 3 * NBUF]
        gat_sems = scratch[3 * NBUF : 4 * NBUF]
        out_sems = scratch[4 * NBUF : 5 * NBUF]

        wid = lax.axis_index("s") * NC + lax.axis_index("c")
        base = wid * B_PER_W

        def wait_idx(b, off):
            pltpu.make_async_copy(
                idx_hbm.at[pl.ds(off, CHUNK)], idx_bufs[b], idx_sems[b]
            ).wait()

        def start_gather(b):
            pltpu.async_copy(table_hbm.at[idx_bufs[b]], row_bufs[b], gat_sems[b])

        def wait_gather(b):
            pltpu.make_async_copy(
                table_hbm.at[idx_bufs[b]], row_bufs[b], gat_sems[b]
            ).wait()

        def start_write(b, off):
            pltpu.async_copy(row_bufs[b], out_hbm.at[pl.ds(off, CHUNK)], out_sems[b])

        def wait_write(b, off):
            pltpu.make_async_copy(
                row_bufs[b], out_hbm.at[pl.ds(off, CHUNK)], out_sems[b]
            ).wait()

        def start_idx(b, off):
            pltpu.async_copy(idx_hbm.at[pl.ds(off, CHUNK)], idx_bufs[b], idx_sems[b])

        for b in range(NBUF):
            start_idx(b, base + b * CHUNK)

        def body(go, carry):
            for b in range(NBUF):
                off = base + (go * NBUF + b) * CHUNK
                wait_idx(b, off)

                @pl.when(go > 0)
                def _():
                    wait_write(b, off - NBUF * CHUNK)

                start_gather(b)

                pb = (b - 1) % NBUF
                poff = off - CHUNK

                def drain_prev():
                    wait_gather(pb)

                    @pl.when(poff + NBUF * CHUNK < base + B_PER_W)
                    def _():
                        start_idx(pb, poff + NBUF * CHUNK)

                    start_write(pb, poff)

                if b == 0:
                    pl.when(go > 0)(drain_prev)
                else:
                    drain_prev()
            return carry

        lax.fori_loop(0, NOUTER, body, 0)

        last = NBUF - 1
        wait_gather(last)
        start_write(last, base + (NCH - 1) * CHUNK)
        for b in range(NBUF):
            wait_write(b, base + (NCH - NBUF + b) * CHUNK)

    return k(idx, table)


def kernel(edge_types, table):
    idx = edge_types.reshape(B).astype(jnp.int32)
    out = _lookup(idx, table)
    return out.reshape(ROWS, COLS, EMBED_DIM)
